# baseline (speedup 1.0000x reference)
import functools
import math

import jax
import jax.numpy as jnp
import numpy as np
from jax import lax
from jax.experimental import pallas as pl
from jax.experimental.pallas import tpu as pltpu
from jax.experimental.pallas import tpu_sc as plsc

B, T = 8, 2048
IN_SIZE = 256
HID = 1024
VOCAB = 16384
MAX_SEQ = 4096
FREQ = 256

N_TOK = B * T
N_PHASE = 2
PH_TOK = N_TOK // N_PHASE
NC, NS = 2, 16
NW = NC * NS
TOK_PER_W = PH_TOK // NW
CHUNK = 32
N_CHUNKS = TOK_PER_W // CHUNK
LANES = 16


def _sincos_table_np():
    half = HID // 2
    scale = math.log(10000.0) / (half - 1)
    freqs = np.exp(np.arange(half, dtype=np.float64) * -scale)
    pos = np.arange(MAX_SEQ + 1, dtype=np.float64)[:, None] * freqs[None, :]
    emb = np.concatenate([np.sin(pos), np.cos(pos)], axis=1).astype(np.float32)
    emb[0, :] = 0.0
    return emb


_SINCOS = _sincos_table_np()


def _sc_gather_body(phase_base, cidx_hbm, pidx_hbm, ttab_hbm, ptab_hbm,
                    emb_out, cidx_v, pidx_v, crows_v, prows_v, csem, psem):
    wid = lax.axis_index("c") * NS + lax.axis_index("s")
    base = phase_base + wid * TOK_PER_W

    def body(i, carry):
        off = base + i * CHUNK
        pltpu.sync_copy(cidx_hbm.at[pl.ds(off, CHUNK)], cidx_v)
        pltpu.sync_copy(pidx_hbm.at[pl.ds(off, CHUNK)], pidx_v)
        cp1 = pltpu.async_copy(ttab_hbm.at[cidx_v], crows_v, csem)
        cp2 = pltpu.async_copy(ptab_hbm.at[pidx_v], prows_v, psem)
        cp1.wait()
        cp2.wait()

        def row_add(r, c2):
            for k in range(HID // LANES):
                v = crows_v[r, pl.ds(k * LANES, LANES)]
                plsc.addupdate(prows_v.at[r, pl.ds(k * LANES, LANES)], v)
            return c2

        lax.fori_loop(0, CHUNK, row_add, 0)
        pltpu.sync_copy(prows_v, emb_out.at[pl.ds(off - phase_base, CHUNK)])
        return carry

    lax.fori_loop(0, N_CHUNKS, body, 0)


def _sc_gather(phase, cidx, pidx, ttab, ptab):
    mesh = plsc.VectorSubcoreMesh(core_axis_name="c", subcore_axis_name="s")
    f = pl.kernel(
        functools.partial(_sc_gather_body, phase * PH_TOK),
        mesh=mesh,
        out_type=jax.ShapeDtypeStruct((PH_TOK, HID), jnp.float32),
        scratch_types=[
            pltpu.VMEM((CHUNK,), jnp.int32),
            pltpu.VMEM((CHUNK,), jnp.int32),
            pltpu.VMEM((CHUNK, HID), jnp.float32),
            pltpu.VMEM((CHUNK, HID), jnp.float32),
            pltpu.SemaphoreType.DMA,
            pltpu.SemaphoreType.DMA,
        ],
    )
    return f(cidx, pidx, ttab, ptab)


def _temb_body(targ_ref, w1_ref, b1_ref, w2_ref, b2_ref, o_ref):
    a = targ_ref[...]
    tf = jnp.concatenate([jnp.cos(a), jnp.sin(a)], axis=1)
    h1 = jnp.dot(tf, w1_ref[...], preferred_element_type=jnp.float32,
                 precision=lax.Precision.HIGHEST) + b1_ref[...]
    h1 = h1 * jax.nn.sigmoid(h1)
    o_ref[...] = jnp.dot(h1, w2_ref[...], preferred_element_type=jnp.float32,
                         precision=lax.Precision.HIGHEST) + b2_ref[...]


TOK_BLK = 512
PH_GRID = PH_TOK // TOK_BLK
BLK_PER_BATCH = T // TOK_BLK


def _main_body(x_ref, w_ref, b_ref, temb_ref, emb_ref, *rest):
    o_ref = rest[-1]
    h = jnp.dot(x_ref[...], w_ref[...], preferred_element_type=jnp.float32,
                precision=lax.Precision.HIGHEST)
    o_ref[...] = h + b_ref[...] + temb_ref[0] + emb_ref[...]


def _main_phase(phase, xf, W_in, b_in2, temb3, emb, prev_out):
    blk0 = phase * PH_GRID

    def xmap(i):
        return (blk0 + i, 0)

    def tmap(i):
        return ((blk0 + i) // BLK_PER_BATCH, 0, 0)

    in_specs = [
        pl.BlockSpec((TOK_BLK, IN_SIZE), xmap),
        pl.BlockSpec((IN_SIZE, HID), lambda i: (0, 0)),
        pl.BlockSpec((1, HID), lambda i: (0, 0)),
        pl.BlockSpec((1, 1, HID), tmap),
        pl.BlockSpec((TOK_BLK, HID), lambda i: (i, 0)),
    ]
    args = [xf, W_in, b_in2, temb3, emb]
    aliases = {}
    if prev_out is not None:
        in_specs.append(pl.BlockSpec(memory_space=pl.ANY))
        args.append(prev_out)
        aliases = {5: 0}
    return pl.pallas_call(
        _main_body,
        grid=(PH_GRID,),
        in_specs=in_specs,
        out_specs=pl.BlockSpec((TOK_BLK, HID), xmap),
        out_shape=jax.ShapeDtypeStruct((N_TOK, HID), jnp.float32),
        input_output_aliases=aliases,
        compiler_params=pltpu.CompilerParams(
            dimension_semantics=("arbitrary",)),
    )(*args)


def kernel(x, position_ids, t, condition, token_table, W_in, b_in,
           W_t1, b_t1, W_t2, b_t2):
    xf = x.reshape(N_TOK, IN_SIZE)
    cond_flat = condition.reshape(N_TOK)
    pos_flat = position_ids.reshape(N_TOK)
    ptab = jnp.asarray(_SINCOS)

    gathered = [
        _sc_gather(p, cond_flat, pos_flat, token_table, ptab)
        for p in range(N_PHASE)
    ]

    half_f = FREQ // 2
    tfreqs = jnp.exp(-math.log(10000.0)
                     * jnp.arange(half_f, dtype=jnp.float32) / half_f)
    targs = t[:, None] * tfreqs[None, :]
    temb = pl.pallas_call(
        _temb_body,
        out_shape=jax.ShapeDtypeStruct((B, HID), jnp.float32),
    )(targs, W_t1, b_t1.reshape(1, HID), W_t2, b_t2.reshape(1, HID))

    b_in2 = b_in.reshape(1, HID)
    temb3 = temb.reshape(B, 1, HID)

    out = None
    for p in range(N_PHASE):
        out = _main_phase(p, xf, W_in, b_in2, temb3, gathered[p], out)

    return out.reshape(B, T, HID)


# speedup vs baseline: 1.2287x; 1.2287x over previous
import functools
import math

import jax
import jax.numpy as jnp
import numpy as np
from jax import lax
from jax.experimental import pallas as pl
from jax.experimental.pallas import tpu as pltpu
from jax.experimental.pallas import tpu_sc as plsc

B, T = 8, 2048
IN_SIZE = 256
HID = 1024
VOCAB = 16384
MAX_SEQ = 4096
FREQ = 256

N_TOK = B * T
N_PHASE = 2
PH_TOK = N_TOK // N_PHASE
NC, NS = 2, 16
NW = NC * NS
TOK_PER_W = PH_TOK // NW
CHUNK = 16
N_CHUNKS = TOK_PER_W // CHUNK
LANES = 16


def _sincos_table_np():
    half = HID // 2
    scale = math.log(10000.0) / (half - 1)
    freqs = np.exp(np.arange(half, dtype=np.float64) * -scale)
    pos = np.arange(MAX_SEQ + 1, dtype=np.float64)[:, None] * freqs[None, :]
    emb = np.concatenate([np.sin(pos), np.cos(pos)], axis=1).astype(np.float32)
    emb[0, :] = 0.0
    return emb


_SINCOS = _sincos_table_np()


def _sc_gather_body(phase_base, cidx_hbm, pidx_hbm, ttab_hbm, ptab_hbm,
                    emb_out,
                    cidx_a, pidx_a, crows_a, prows_a,
                    cidx_b, pidx_b, crows_b, prows_b,
                    csem_a, psem_a, csem_b, psem_b):
    wid = lax.axis_index("c") * NS + lax.axis_index("s")
    base = phase_base + wid * TOK_PER_W

    def issue(off, cidx_v, pidx_v, crows_v, prows_v, csem, psem):
        pltpu.sync_copy(cidx_hbm.at[pl.ds(off, CHUNK)], cidx_v)
        pltpu.sync_copy(pidx_hbm.at[pl.ds(off, CHUNK)], pidx_v)
        pltpu.async_copy(ttab_hbm.at[cidx_v], crows_v, csem)
        pltpu.async_copy(ptab_hbm.at[pidx_v], prows_v, psem)

    def drain(off, cidx_v, pidx_v, crows_v, prows_v, csem, psem):
        pltpu.make_async_copy(ttab_hbm.at[cidx_v], crows_v, csem).wait()
        pltpu.make_async_copy(ptab_hbm.at[pidx_v], prows_v, psem).wait()

        def row_add(r, c2):
            for k in range(HID // LANES):
                v = crows_v[r, pl.ds(k * LANES, LANES)]
                plsc.addupdate(prows_v.at[r, pl.ds(k * LANES, LANES)], v)
            return c2

        lax.fori_loop(0, CHUNK, row_add, 0)
        pltpu.sync_copy(prows_v, emb_out.at[pl.ds(off - phase_base, CHUNK)])

    bufs_a = (cidx_a, pidx_a, crows_a, prows_a, csem_a, psem_a)
    bufs_b = (cidx_b, pidx_b, crows_b, prows_b, csem_b, psem_b)

    issue(base, *bufs_a)
    n_pair = N_CHUNKS // 2

    def body(j, carry):
        off_a = base + (2 * j) * CHUNK
        off_b = off_a + CHUNK
        issue(off_b, *bufs_b)
        drain(off_a, *bufs_a)

        @pl.when(j + 1 < n_pair)
        def _():
            issue(off_b + CHUNK, *bufs_a)

        drain(off_b, *bufs_b)
        return carry

    lax.fori_loop(0, n_pair, body, 0)


def _sc_gather(phase, cidx, pidx, ttab, ptab):
    mesh = plsc.VectorSubcoreMesh(core_axis_name="c", subcore_axis_name="s")
    f = pl.kernel(
        functools.partial(_sc_gather_body, phase * PH_TOK),
        mesh=mesh,
        out_type=jax.ShapeDtypeStruct((PH_TOK, HID), jnp.float32),
        scratch_types=[
            pltpu.VMEM((CHUNK,), jnp.int32),
            pltpu.VMEM((CHUNK,), jnp.int32),
            pltpu.VMEM((CHUNK, HID), jnp.float32),
            pltpu.VMEM((CHUNK, HID), jnp.float32),
            pltpu.VMEM((CHUNK,), jnp.int32),
            pltpu.VMEM((CHUNK,), jnp.int32),
            pltpu.VMEM((CHUNK, HID), jnp.float32),
            pltpu.VMEM((CHUNK, HID), jnp.float32),
            pltpu.SemaphoreType.DMA,
            pltpu.SemaphoreType.DMA,
            pltpu.SemaphoreType.DMA,
            pltpu.SemaphoreType.DMA,
        ],
    )
    return f(cidx, pidx, ttab, ptab)


def _temb_body(targ_ref, w1_ref, b1_ref, w2_ref, b2_ref, o_ref):
    a = targ_ref[...]
    tf = jnp.concatenate([jnp.cos(a), jnp.sin(a)], axis=1)
    h1 = jnp.dot(tf, w1_ref[...], preferred_element_type=jnp.float32,
                 precision=lax.Precision.HIGHEST) + b1_ref[...]
    h1 = h1 * jax.nn.sigmoid(h1)
    o_ref[...] = jnp.dot(h1, w2_ref[...], preferred_element_type=jnp.float32,
                         precision=lax.Precision.HIGHEST) + b2_ref[...]


TOK_BLK = 512
PH_GRID = PH_TOK // TOK_BLK
BLK_PER_BATCH = T // TOK_BLK


def _main_body(x_ref, w_ref, b_ref, temb_ref, emb_ref, *rest):
    o_ref = rest[-1]
    h = jnp.dot(x_ref[...], w_ref[...], preferred_element_type=jnp.float32,
                precision=lax.Precision.HIGHEST)
    o_ref[...] = h + b_ref[...] + temb_ref[0] + emb_ref[...]


def _main_phase(phase, xf, W_in, b_in2, temb3, emb, prev_out):
    blk0 = phase * PH_GRID

    def xmap(i):
        return (blk0 + i, 0)

    def tmap(i):
        return ((blk0 + i) // BLK_PER_BATCH, 0, 0)

    in_specs = [
        pl.BlockSpec((TOK_BLK, IN_SIZE), xmap),
        pl.BlockSpec((IN_SIZE, HID), lambda i: (0, 0)),
        pl.BlockSpec((1, HID), lambda i: (0, 0)),
        pl.BlockSpec((1, 1, HID), tmap),
        pl.BlockSpec((TOK_BLK, HID), lambda i: (i, 0)),
    ]
    args = [xf, W_in, b_in2, temb3, emb]
    aliases = {}
    if prev_out is not None:
        in_specs.append(pl.BlockSpec(memory_space=pl.ANY))
        args.append(prev_out)
        aliases = {5: 0}
    return pl.pallas_call(
        _main_body,
        grid=(PH_GRID,),
        in_specs=in_specs,
        out_specs=pl.BlockSpec((TOK_BLK, HID), xmap),
        out_shape=jax.ShapeDtypeStruct((N_TOK, HID), jnp.float32),
        input_output_aliases=aliases,
        compiler_params=pltpu.CompilerParams(
            dimension_semantics=("arbitrary",)),
    )(*args)


def kernel(x, position_ids, t, condition, token_table, W_in, b_in,
           W_t1, b_t1, W_t2, b_t2):
    xf = x.reshape(N_TOK, IN_SIZE)
    cond_flat = condition.reshape(N_TOK)
    pos_flat = position_ids.reshape(N_TOK)
    ptab = jnp.asarray(_SINCOS) * (t[0] * 0.0 + 1.0)

    gathered = [
        _sc_gather(p, cond_flat, pos_flat, token_table, ptab)
        for p in range(N_PHASE)
    ]

    half_f = FREQ // 2
    tfreqs = jnp.exp(-math.log(10000.0)
                     * jnp.arange(half_f, dtype=jnp.float32) / half_f)
    targs = t[:, None] * tfreqs[None, :]
    temb = pl.pallas_call(
        _temb_body,
        out_shape=jax.ShapeDtypeStruct((B, HID), jnp.float32),
    )(targs, W_t1, b_t1.reshape(1, HID), W_t2, b_t2.reshape(1, HID))

    b_in2 = b_in.reshape(1, HID)
    temb3 = temb.reshape(B, 1, HID)

    out = None
    for p in range(N_PHASE):
        out = _main_phase(p, xf, W_in, b_in2, temb3, gathered[p], out)

    return out.reshape(B, T, HID)
